# baseline (speedup 1.0000x reference)
import functools

import jax
import jax.numpy as jnp
from jax import lax
from jax.experimental import pallas as pl
from jax.experimental.pallas import tpu as pltpu
from jax.experimental.pallas import tpu_sc as plsc

NC = 2
NS = 16
NW = NC * NS
LANES = 16
BLK = 128
WIN = 512


def _make_tc_pack(vocab, dim):
    grid = (vocab + WIN - 1) // WIN
    half = WIN // 2

    def body(in_ref, o_ref):
        a = in_ref[:, pl.ds(0, half)]
        b = in_ref[:, pl.ds(half, half)]
        o_ref[...] = jnp.concatenate([a.T, b.T], axis=1)

    return pl.pallas_call(
        body,
        grid=(grid,),
        in_specs=[pl.BlockSpec((dim, WIN), lambda w: (0, w))],
        out_specs=pl.BlockSpec((half, 2 * dim), lambda w: (w, 0)),
        out_shape=jax.ShapeDtypeStruct((grid * half, 2 * dim), jnp.float32),
    )


def _make_sc_kernel(batch, seq_len, dim):
    assert batch == NW * BLK and dim % LANES == 0
    n_vregs = dim // LANES

    mesh = plsc.VectorSubcoreMesh(core_axis_name="c", subcore_axis_name="s")

    @functools.partial(
        pl.kernel,
        out_type=jax.ShapeDtypeStruct((seq_len * dim, batch), jnp.float32),
        mesh=mesh,
        scratch_types=[
            [pltpu.VMEM((1, BLK), jnp.int32) for _ in range(4)],
            [pltpu.VMEM((BLK,), jnp.int32) for _ in range(4)],
            [pltpu.VMEM((BLK, 2 * dim), jnp.float32) for _ in range(4)],
            [pltpu.VMEM((dim, BLK), jnp.float32) for _ in range(2)],
            pltpu.VMEM((seq_len, dim), jnp.float32),
            [pltpu.SemaphoreType.DMA for _ in range(4)],
            [pltpu.SemaphoreType.DMA for _ in range(4)],
            [pltpu.SemaphoreType.DMA for _ in range(2)],
        ],
        compiler_params=pltpu.CompilerParams(
            use_tc_tiling_on_sc=True, needs_layout_passes=False),
    )
    def sc_kernel(idx_hbm, tok_hbm, pos_hbm, out_hbm,
                  idx, idx2, rows, out_t, pos_v, isem, gsem, ssem):
        wid = lax.axis_index("s") * NC + lax.axis_index("c")
        b0 = wid * BLK

        pltpu.sync_copy(pos_hbm, pos_v)

        def idx_start(s, p):
            pltpu.async_copy(idx_hbm.at[pl.ds(s, 1), pl.ds(b0, BLK)],
                             idx[p], isem[p])

        def idx_wait(s, p):
            pltpu.make_async_copy(idx_hbm.at[pl.ds(s, 1), pl.ds(b0, BLK)],
                                  idx[p], isem[p]).wait()

        def fetch_start(s, p):
            idx_wait(s, p)
            for g in range(BLK // LANES):
                sl = pl.ds(g * LANES, LANES)
                v = idx[p][0, sl]
                idx2[p][sl] = lax.shift_left(
                    lax.shift_right_logical(v, 9), 8) | (v & (WIN // 2 - 1))
            pltpu.async_copy(tok_hbm.at[idx2[p]], rows[p], gsem[p])

        def fetch_wait(p):
            pltpu.make_async_copy(tok_hbm.at[idx2[p]], rows[p], gsem[p]).wait()

        def store_start(s, po):
            pltpu.async_copy(
                out_t[po], out_hbm.at[pl.ds(s * dim, dim), pl.ds(b0, BLK)],
                ssem[po])

        def store_wait(s, po):
            pltpu.make_async_copy(
                out_t[po], out_hbm.at[pl.ds(s * dim, dim), pl.ds(b0, BLK)],
                ssem[po]).wait()

        def process(s, q):
            p, po = q, q % 2
            fetch_wait(p)
            s_splat = jnp.full((LANES,), s, jnp.int32)
            iota = jnp.arange(LANES, dtype=jnp.int32)
            jvecs = [iota + g * LANES for g in range(BLK // LANES)]
            hvecs = [
                (lax.shift_right_logical(idx[p][0, pl.ds(g * LANES, LANES)], 8)
                 & 1) * dim
                for g in range(BLK // LANES)
            ]

            def k_body(k, carry):
                rot = (iota + k) & (LANES - 1)
                for c in range(n_vregs):
                    dvec = rot + c * LANES
                    pd = plsc.load_gather(pos_v, [s_splat, dvec])
                    for g in range(BLK // LANES):
                        val = plsc.load_gather(
                            rows[p], [jvecs[g], hvecs[g] + dvec]) + pd
                        plsc.store_scatter(out_t[po], [dvec, jvecs[g]], val)
                return carry

            lax.fori_loop(0, LANES, k_body, 0)
            store_start(s, po)

        idx_start(0, 0)
        idx_start(1, 1)
        idx_start(2, 2)
        fetch_start(0, 0)
        fetch_start(1, 1)

        def loop_body(t, carry):
            for q in range(4):
                s = 4 * t + q
                pl.when(s + 3 < seq_len)(
                    functools.partial(idx_start, s + 3, (q + 3) % 4))
                pl.when(s + 2 < seq_len)(
                    functools.partial(fetch_start, s + 2, (q + 2) % 4))
                pl.when(s >= 2)(functools.partial(store_wait, s - 2, q % 2))
                process(s, q)
            return carry

        lax.fori_loop(0, seq_len // 4, loop_body, 0)

        store_wait(seq_len - 2, 0)
        store_wait(seq_len - 1, 1)

    return sc_kernel


def kernel(inputs, token_table, position_table):
    batch, seq_len = inputs.shape
    vocab, dim = token_table.shape
    idx_t = inputs.T
    tok_t = token_table.T
    tok2 = _make_tc_pack(vocab, dim)(tok_t)
    sc = _make_sc_kernel(batch, seq_len, dim)
    out = sc(idx_t, tok2, position_table)
    return out.reshape(seq_len, dim, batch).transpose(2, 0, 1)


# speedup vs baseline: 1.6888x; 1.6888x over previous
import functools

import jax
import jax.numpy as jnp
from jax import lax
from jax.experimental import pallas as pl
from jax.experimental.pallas import tpu as pltpu
from jax.experimental.pallas import tpu_sc as plsc

NC = 2
NS = 16
NW = NC * NS
LANES = 16
BLK = 128


def _make_sc_pack(vocab, dim):
    full_win = vocab // BLK
    tail_n = vocab - full_win * BLK
    n_rows = full_win * (BLK // 2) + (tail_n + 1) // 2
    per_w = (full_win + NW - 1) // NW
    n_vregs = dim // LANES

    mesh = plsc.VectorSubcoreMesh(core_axis_name="c", subcore_axis_name="s")

    @functools.partial(
        pl.kernel,
        out_type=jax.ShapeDtypeStruct((n_rows, 2 * dim), jnp.float32),
        mesh=mesh,
        scratch_types=[
            [pltpu.VMEM((dim, BLK), jnp.float32) for _ in range(2)],
            [pltpu.VMEM((BLK // 2, 2 * dim), jnp.float32) for _ in range(2)],
            pltpu.VMEM((dim, max(tail_n, LANES)), jnp.float32)
            if tail_n else None,
            [pltpu.SemaphoreType.DMA for _ in range(2)],
            [pltpu.SemaphoreType.DMA for _ in range(2)],
        ],
        compiler_params=pltpu.CompilerParams(
            use_tc_tiling_on_sc=True, needs_layout_passes=False),
    )
    def pack_kernel(tok_hbm, tail_hbm, out_hbm, in_v, out_v, tail_v, gsem, ssem):
        wid = lax.axis_index("s") * NC + lax.axis_index("c")

        def win_of(u):
            return wid + NW * u

        def fetch_start(u, p):
            pltpu.async_copy(tok_hbm.at[:, pl.ds(win_of(u) * BLK, BLK)],
                             in_v[p], gsem[p])

        def fetch_wait(u, p):
            pltpu.make_async_copy(tok_hbm.at[:, pl.ds(win_of(u) * BLK, BLK)],
                                  in_v[p], gsem[p]).wait()

        def store_start(u, p):
            pltpu.async_copy(
                out_v[p],
                out_hbm.at[pl.ds(win_of(u) * (BLK // 2), BLK // 2)], ssem[p])

        def store_wait(u, p):
            pltpu.make_async_copy(
                out_v[p],
                out_hbm.at[pl.ds(win_of(u) * (BLK // 2), BLK // 2)],
                ssem[p]).wait()

        iota = jnp.arange(LANES, dtype=jnp.int32)
        jvecs = [iota + g * LANES for g in range(BLK // LANES)]
        rvecs = [jv >> 1 for jv in jvecs]
        hvecs = [(jv & 1) * dim for jv in jvecs]

        def process(u, p):
            fetch_wait(u, p)

            def k_body(k, carry):
                rot = (iota + k) & (LANES - 1)
                for c in range(n_vregs):
                    dvec = rot + c * LANES
                    for g in range(BLK // LANES):
                        val = plsc.load_gather(in_v[p], [dvec, jvecs[g]])
                        plsc.store_scatter(out_v[p], [rvecs[g], hvecs[g] + dvec],
                                           val)
                return carry

            lax.fori_loop(0, LANES, k_body, 0)
            store_start(u, p)

        fetch_start(0, 0)

        def loop_body(t, carry):
            for par in range(2):
                u = 2 * t + par
                pl.when((u + 1 < per_w) & (win_of(u + 1) < full_win))(
                    functools.partial(fetch_start, u + 1, 1 - par))
                pl.when(u >= 2)(functools.partial(store_wait, u - 2, par))
                pl.when(win_of(u) < full_win)(functools.partial(process, u, par))
            return carry

        lax.fori_loop(0, (per_w + 1) // 2, loop_body, 0)

        last = 2 * ((per_w + 1) // 2) - 3
        for u in range(max(last + 1, 0), per_w):
            pl.when(win_of(u) < full_win)(
                functools.partial(store_wait, u, u % 2))

        if tail_n:
            def do_tail():
                pltpu.sync_copy(tail_hbm, tail_v)

                def tk_body(k, carry):
                    rot = (iota + k) & (LANES - 1)
                    for c in range(n_vregs):
                        dvec = rot + c * LANES
                        for g in range(tail_n // LANES):
                            val = plsc.load_gather(tail_v, [dvec, jvecs[g]])
                            plsc.store_scatter(
                                out_v[0], [rvecs[g], hvecs[g] + dvec], val)
                    return carry

                lax.fori_loop(0, LANES, tk_body, 0)
                pltpu.sync_copy(
                    out_v[0].at[pl.ds(0, tail_n // 2)],
                    out_hbm.at[pl.ds(full_win * (BLK // 2), tail_n // 2)])

            pl.when(wid == 0)(do_tail)

    return pack_kernel


def _make_sc_lookup(batch, seq_len, dim):
    assert batch == NW * BLK and dim % LANES == 0
    n_vregs = dim // LANES

    mesh = plsc.VectorSubcoreMesh(core_axis_name="c", subcore_axis_name="s")

    @functools.partial(
        pl.kernel,
        out_type=jax.ShapeDtypeStruct((seq_len * dim, batch), jnp.float32),
        mesh=mesh,
        scratch_types=[
            [pltpu.VMEM((1, BLK), jnp.int32) for _ in range(4)],
            [pltpu.VMEM((BLK,), jnp.int32) for _ in range(4)],
            [pltpu.VMEM((BLK, 2 * dim), jnp.float32) for _ in range(4)],
            [pltpu.VMEM((dim, BLK), jnp.float32) for _ in range(2)],
            pltpu.VMEM((seq_len, dim), jnp.float32),
            [pltpu.SemaphoreType.DMA for _ in range(4)],
            [pltpu.SemaphoreType.DMA for _ in range(4)],
            [pltpu.SemaphoreType.DMA for _ in range(2)],
        ],
        compiler_params=pltpu.CompilerParams(
            use_tc_tiling_on_sc=True, needs_layout_passes=False),
    )
    def sc_kernel(idx_hbm, tok_hbm, pos_hbm, out_hbm,
                  idx, idx2, rows, out_t, pos_v, isem, gsem, ssem):
        wid = lax.axis_index("s") * NC + lax.axis_index("c")
        b0 = wid * BLK

        pltpu.sync_copy(pos_hbm, pos_v)

        def idx_start(s, p):
            pltpu.async_copy(idx_hbm.at[pl.ds(s, 1), pl.ds(b0, BLK)],
                             idx[p], isem[p])

        def idx_wait(s, p):
            pltpu.make_async_copy(idx_hbm.at[pl.ds(s, 1), pl.ds(b0, BLK)],
                                  idx[p], isem[p]).wait()

        def fetch_start(s, p):
            idx_wait(s, p)
            for g in range(BLK // LANES):
                sl = pl.ds(g * LANES, LANES)
                idx2[p][sl] = lax.shift_right_logical(idx[p][0, sl], 1)
            pltpu.async_copy(tok_hbm.at[idx2[p]], rows[p], gsem[p])

        def fetch_wait(p):
            pltpu.make_async_copy(tok_hbm.at[idx2[p]], rows[p], gsem[p]).wait()

        def store_start(s, po):
            pltpu.async_copy(
                out_t[po], out_hbm.at[pl.ds(s * dim, dim), pl.ds(b0, BLK)],
                ssem[po])

        def store_wait(s, po):
            pltpu.make_async_copy(
                out_t[po], out_hbm.at[pl.ds(s * dim, dim), pl.ds(b0, BLK)],
                ssem[po]).wait()

        def process(s, q):
            p, po = q, q % 2
            fetch_wait(p)
            s_splat = jnp.full((LANES,), s, jnp.int32)
            iota = jnp.arange(LANES, dtype=jnp.int32)
            jvecs = [iota + g * LANES for g in range(BLK // LANES)]
            hvecs = [(idx[p][0, pl.ds(g * LANES, LANES)] & 1) * dim
                     for g in range(BLK // LANES)]

            def k_body(k, carry):
                rot = (iota + k) & (LANES - 1)
                for c in range(n_vregs):
                    dvec = rot + c * LANES
                    pd = plsc.load_gather(pos_v, [s_splat, dvec])
                    for g in range(BLK // LANES):
                        val = plsc.load_gather(
                            rows[p], [jvecs[g], hvecs[g] + dvec]) + pd
                        plsc.store_scatter(out_t[po], [dvec, jvecs[g]], val)
                return carry

            lax.fori_loop(0, LANES, k_body, 0)
            store_start(s, po)

        idx_start(0, 0)
        idx_start(1, 1)
        idx_start(2, 2)
        fetch_start(0, 0)
        fetch_start(1, 1)

        def loop_body(t, carry):
            for q in range(4):
                s = 4 * t + q
                pl.when(s + 3 < seq_len)(
                    functools.partial(idx_start, s + 3, (q + 3) % 4))
                pl.when(s + 2 < seq_len)(
                    functools.partial(fetch_start, s + 2, (q + 2) % 4))
                pl.when(s >= 2)(functools.partial(store_wait, s - 2, q % 2))
                process(s, q)
            return carry

        lax.fori_loop(0, seq_len // 4, loop_body, 0)

        store_wait(seq_len - 2, 0)
        store_wait(seq_len - 1, 1)

    return sc_kernel


def kernel(inputs, token_table, position_table):
    batch, seq_len = inputs.shape
    vocab, dim = token_table.shape
    idx_t = inputs.T
    tok_t = token_table.T
    tail = tok_t[:, (vocab // BLK) * BLK:]
    tok2 = _make_sc_pack(vocab, dim)(tok_t, tail)
    sc = _make_sc_lookup(batch, seq_len, dim)
    out = sc(idx_t, tok2, position_table)
    return out.reshape(seq_len, dim, batch).transpose(2, 0, 1)
